# baseline (speedup 1.0000x reference)
import functools
import jax
import jax.numpy as jnp
from jax import lax
from jax.experimental import pallas as pl
from jax.experimental.pallas import tpu as pltpu
from jax.experimental.pallas import tpu_sc as plsc

_NC = 2
_NS = 16
_NW = _NC * _NS
_C = 16


def kernel(x, pos_table):
    batch, seq, d = x.shape
    per_w = seq // _NW
    chunk = _C * d
    n_chunks = per_w // _C
    x_flat = x.reshape(batch, seq * d)
    pos_flat = pos_table.reshape(-1)[: seq * d]
    mesh = plsc.VectorSubcoreMesh(core_axis_name="c", subcore_axis_name="s")

    @functools.partial(
        pl.kernel,
        out_type=jax.ShapeDtypeStruct((batch, seq * d), jnp.float32),
        mesh=mesh,
        scratch_types=[
            pltpu.VMEM((chunk,), jnp.float32),
            pltpu.VMEM((chunk,), jnp.float32),
            pltpu.SemaphoreType.DMA,
        ],
    )
    def sc_add(x_hbm, pos_hbm, out_hbm, pos_v, x_v, sem):
        wid = lax.axis_index("s") * _NC + lax.axis_index("c")
        base = wid * per_w * d

        @pl.loop(0, n_chunks)
        def _chunks(ci):
            off = base + ci * chunk
            pltpu.sync_copy(pos_hbm.at[pl.ds(off, chunk)], pos_v)
            for b in range(batch):
                pltpu.sync_copy(x_hbm.at[b, pl.ds(off, chunk)], x_v)

                def _add(i, carry):
                    sl = pl.ds(i * 16, 16)
                    x_v[sl] = x_v[sl] + pos_v[sl]
                    return carry

                lax.fori_loop(0, chunk // 16, _add, 0, unroll=4)

                pltpu.sync_copy(x_v, out_hbm.at[b, pl.ds(off, chunk)])

    out = sc_add(x_flat, pos_flat)
    return out.reshape(batch, seq, d)


# speedup vs baseline: 1.2912x; 1.2912x over previous
import functools
import jax
import jax.numpy as jnp
from jax import lax
from jax.experimental import pallas as pl
from jax.experimental.pallas import tpu as pltpu
from jax.experimental.pallas import tpu_sc as plsc

_NC = 2
_NS = 16
_NW = _NC * _NS
_C = 8


def kernel(x, pos_table):
    batch, seq, d = x.shape
    per_w = seq // _NW
    chunk = _C * d
    n_chunks = per_w // _C
    x_flat = x.reshape(batch, seq * d)
    pos_flat = pos_table.reshape(-1)[: seq * d]
    mesh = plsc.VectorSubcoreMesh(core_axis_name="c", subcore_axis_name="s")

    @functools.partial(
        pl.kernel,
        out_type=jax.ShapeDtypeStruct((batch, seq * d), jnp.float32),
        mesh=mesh,
        scratch_types=[
            pltpu.VMEM((2, chunk), jnp.float32),
            pltpu.VMEM((2, batch, chunk), jnp.float32),
            pltpu.SemaphoreType.DMA,
            pltpu.SemaphoreType.DMA,
        ],
    )
    def sc_add(x_hbm, pos_hbm, out_hbm, pos_v, x_v, sem_in, sem_out):
        wid = lax.axis_index("s") * _NC + lax.axis_index("c")
        base = wid * per_w * d

        def issue_in(ci, par):
            off = base + ci * chunk
            pltpu.async_copy(pos_hbm.at[pl.ds(off, chunk)], pos_v.at[par], sem_in)
            for b in range(batch):
                pltpu.async_copy(
                    x_hbm.at[b, pl.ds(off, chunk)], x_v.at[par, b], sem_in
                )

        def wait_in(par):
            pltpu.make_async_copy(
                pos_hbm.at[pl.ds(0, chunk)], pos_v.at[par], sem_in
            ).wait()
            for b in range(batch):
                pltpu.make_async_copy(
                    x_hbm.at[b, pl.ds(0, chunk)], x_v.at[par, b], sem_in
                ).wait()

        def issue_out(ci, par):
            off = base + ci * chunk
            for b in range(batch):
                pltpu.async_copy(
                    x_v.at[par, b], out_hbm.at[b, pl.ds(off, chunk)], sem_out
                )

        def wait_out(par):
            for b in range(batch):
                pltpu.make_async_copy(
                    x_v.at[par, b], out_hbm.at[b, pl.ds(0, chunk)], sem_out
                ).wait()

        issue_in(0, 0)

        @pl.loop(0, n_chunks, step=2)
        def _chunks(ci0):
            for par in range(2):
                ci = ci0 + par
                opp = 1 - par

                if par == 0:
                    @pl.when(ci0 >= 1)
                    def _():
                        wait_out(opp)

                    issue_in(ci + 1, opp)
                else:
                    wait_out(opp)

                    @pl.when(ci0 < n_chunks - 2)
                    def _():
                        issue_in(ci + 1, opp)

                wait_in(par)

                def _add(i, carry):
                    sl = pl.ds(i * 16, 16)
                    pv = pos_v[par, sl]
                    for b in range(batch):
                        x_v[par, b, sl] = x_v[par, b, sl] + pv
                    return carry

                lax.fori_loop(0, chunk // 16, _add, 0, unroll=4)
                issue_out(ci, par)

        wait_out(1)

    out = sc_add(x_flat, pos_flat)
    return out.reshape(batch, seq, d)


# speedup vs baseline: 4.1433x; 3.2089x over previous
import functools
import jax
import jax.numpy as jnp
from jax import lax
from jax.experimental import pallas as pl
from jax.experimental.pallas import tpu as pltpu
from jax.experimental.pallas import tpu_sc as plsc

_NC = 2
_NS = 16
_NW = _NC * _NS
_C = 8


def kernel(x, pos_table):
    batch, seq, d = x.shape
    per_w = seq // _NW
    n_chunks = per_w // _C
    mesh = plsc.VectorSubcoreMesh(core_axis_name="c", subcore_axis_name="s")

    @functools.partial(
        pl.kernel,
        out_type=jax.ShapeDtypeStruct((batch, seq, d), jnp.float32),
        mesh=mesh,
        scratch_types=[
            pltpu.VMEM((2, _C, d), jnp.float32),
            pltpu.VMEM((2, batch, _C, d), jnp.float32),
            pltpu.SemaphoreType.DMA,
            pltpu.SemaphoreType.DMA,
        ],
        compiler_params=pltpu.CompilerParams(use_tc_tiling_on_sc=True),
    )
    def sc_add(x_hbm, pos_hbm, out_hbm, pos_v, x_v, sem_in, sem_out):
        wid = lax.axis_index("s") * _NC + lax.axis_index("c")
        base = wid * per_w

        def issue_in(ci, par):
            s0 = base + ci * _C
            pltpu.async_copy(pos_hbm.at[pl.ds(s0, _C), :], pos_v.at[par], sem_in)
            for b in range(batch):
                pltpu.async_copy(
                    x_hbm.at[b, pl.ds(s0, _C), :], x_v.at[par, b], sem_in
                )

        def wait_in(par):
            pltpu.make_async_copy(
                pos_hbm.at[pl.ds(0, _C), :], pos_v.at[par], sem_in
            ).wait()
            for b in range(batch):
                pltpu.make_async_copy(
                    x_hbm.at[b, pl.ds(0, _C), :], x_v.at[par, b], sem_in
                ).wait()

        def issue_out(ci, par):
            s0 = base + ci * _C
            for b in range(batch):
                pltpu.async_copy(
                    x_v.at[par, b], out_hbm.at[b, pl.ds(s0, _C), :], sem_out
                )

        def wait_out(par):
            for b in range(batch):
                pltpu.make_async_copy(
                    x_v.at[par, b], out_hbm.at[b, pl.ds(0, _C), :], sem_out
                ).wait()

        issue_in(0, 0)

        @pl.loop(0, n_chunks, step=2)
        def _chunks(ci0):
            for par in range(2):
                ci = ci0 + par
                opp = 1 - par

                if par == 0:
                    @pl.when(ci0 >= 1)
                    def _():
                        wait_out(opp)

                    issue_in(ci + 1, opp)
                else:
                    wait_out(opp)

                    @pl.when(ci0 < n_chunks - 2)
                    def _():
                        issue_in(ci + 1, opp)

                wait_in(par)

                def _add(i, carry):
                    s = i >> 6
                    h = (i & 63) * 16
                    sl = pl.ds(h, 16)
                    pv = pos_v[par, s, sl]
                    for b in range(batch):
                        x_v[par, b, s, sl] = x_v[par, b, s, sl] + pv
                    return carry

                lax.fori_loop(0, _C * (d // 16), _add, 0, unroll=4)
                issue_out(ci, par)

        wait_out(1)

    return sc_add(x, pos_table)


# speedup vs baseline: 5.4649x; 1.3190x over previous
import functools
import jax
import jax.numpy as jnp
from jax import lax
from jax.experimental import pallas as pl
from jax.experimental.pallas import tpu as pltpu
from jax.experimental.pallas import tpu_sc as plsc

_NC = 2
_NS = 16
_NW = _NC * _NS
_C = 8


def kernel(x, pos_table):
    batch, seq, d = x.shape
    per_w = seq // _NW
    n_chunks = per_w // _C
    mesh = plsc.VectorSubcoreMesh(core_axis_name="c", subcore_axis_name="s")

    @functools.partial(
        pl.kernel,
        out_type=jax.ShapeDtypeStruct((batch, seq, d), jnp.float32),
        mesh=mesh,
        scratch_types=[
            pltpu.VMEM((2, _C, d), jnp.float32),
            pltpu.VMEM((2, batch, _C, d), jnp.float32),
            pltpu.SemaphoreType.DMA,
            pltpu.SemaphoreType.DMA,
        ],
        compiler_params=pltpu.CompilerParams(use_tc_tiling_on_sc=True),
    )
    def sc_add(x_hbm, pos_hbm, out_hbm, pos_v, x_v, sem_in, sem_out):
        wid = lax.axis_index("s") * _NC + lax.axis_index("c")
        base = wid * per_w

        def issue_in(ci, par):
            s0 = base + ci * _C
            pltpu.async_copy(pos_hbm.at[pl.ds(s0, _C), :], pos_v.at[par], sem_in)
            for b in range(batch):
                pltpu.async_copy(
                    x_hbm.at[b, pl.ds(s0, _C), :], x_v.at[par, b], sem_in
                )

        def wait_in(par):
            pltpu.make_async_copy(
                pos_hbm.at[pl.ds(0, _C), :], pos_v.at[par], sem_in
            ).wait()
            for b in range(batch):
                pltpu.make_async_copy(
                    x_hbm.at[b, pl.ds(0, _C), :], x_v.at[par, b], sem_in
                ).wait()

        def issue_out(ci, par):
            s0 = base + ci * _C
            for b in range(batch):
                pltpu.async_copy(
                    x_v.at[par, b], out_hbm.at[b, pl.ds(s0, _C), :], sem_out
                )

        def wait_out(par):
            for b in range(batch):
                pltpu.make_async_copy(
                    x_v.at[par, b], out_hbm.at[b, pl.ds(0, _C), :], sem_out
                ).wait()

        issue_in(0, 0)

        @pl.loop(0, n_chunks, step=2)
        def _chunks(ci0):
            for par in range(2):
                ci = ci0 + par
                opp = 1 - par

                if par == 0:
                    @pl.when(ci0 >= 1)
                    def _():
                        wait_out(opp)

                    issue_in(ci + 1, opp)
                else:
                    wait_out(opp)

                    @pl.when(ci0 < n_chunks - 2)
                    def _():
                        issue_in(ci + 1, opp)

                wait_in(par)

                for s in range(_C):
                    def _add(i, carry, s=s):
                        sl = pl.ds(i * 16, 16)
                        pv = pos_v[par, s, sl]
                        for b in range(batch):
                            x_v[par, b, s, sl] = x_v[par, b, s, sl] + pv
                        return carry

                    lax.fori_loop(0, d // 16, _add, 0, unroll=8)
                issue_out(ci, par)

        wait_out(1)

    return sc_add(x, pos_table)
